# baseline (speedup 1.0000x reference)
import jax
import jax.numpy as jnp
from jax import lax
from jax.experimental import pallas as pl
from jax.experimental.pallas import tpu as pltpu
from jax.experimental.pallas import tpu_sc as plsc

BATCH = 32
SIZE = 14
PIX = SIZE * SIZE
CH = 768
NCHUNK = CH // 16
K_PIX = 10
K_CEN = 88
OUT_COLS = K_CEN + PIX * K_PIX
NP = 8
HB = 16
NSLAB = PIX // 16
NXTRA = PIX - 16 * NSLAB


def _sort_desc(v):
    return plsc.sort_key_val(v, v, descending=True)[0]


def _bitonic_desc(c):
    n = len(c)
    if n == 1:
        return [_sort_desc(c[0])]
    h = [jnp.maximum(c[i], c[i + n // 2]) for i in range(n // 2)]
    l = [jnp.minimum(c[i], c[i + n // 2]) for i in range(n // 2)]
    return _bitonic_desc(h) + _bitonic_desc(l)


def _merge(a, b, top_only=False):
    n = len(a)
    rb = [lax.rev(v, (0,)) for v in reversed(b)]
    c = [jnp.maximum(a[i], rb[i]) for i in range(n)]
    top = _bitonic_desc(c)
    if top_only:
        return top
    d = [jnp.minimum(a[i], rb[i]) for i in range(n)]
    return top + _bitonic_desc(d)


def _topk_chains(chunk_v, bsel, b0, np_):
    neg = jnp.full((16,), -jnp.inf, jnp.float32)
    tops = [neg] * np_

    def _asc(c):
        return [
            lax.sort(chunk_v[bsel, 0, b0 + q, pl.ds(c * 16, 16)],
                     dimension=0)
            for q in range(np_)
        ]

    vas = _asc(0)
    for c in range(NCHUNK):
        nxt = _asc(c + 1) if c + 1 < NCHUNK else None
        for q in range(np_):
            tops[q], _u = plsc.sort_key_val(
                jnp.maximum(tops[q], vas[q]), vas[q], descending=True)
        vas = nxt
    return tops


def _sc_body(x_hbm, out4, chunk_v, xchunk_v, cslab_v, stage_v, xstage_v,
             pieces_v, xpieces_v, row_v, row2_v, shared_pix, shared_row,
             shared_extra, dma_sem, xdma_sem, out_sem):
    cid = lax.axis_index("c")
    sid = lax.axis_index("s")
    coff = cid * HB

    start = sid * NSLAB
    pltpu.async_copy(
        x_hbm.at[pl.ds(start // SIZE, 1), pl.ds(lax.rem(start, SIZE), 1),
                 pl.ds(coff, HB)],
        chunk_v.at[pl.ds(0, 1)], dma_sem)
    xslab = 16 * NSLAB + sid // 4
    pltpu.async_copy(
        x_hbm.at[pl.ds(xslab // SIZE, 1), pl.ds(lax.rem(xslab, SIZE), 1),
                 pl.ds(coff, HB)],
        xchunk_v, xdma_sem)
    pltpu.sync_copy(
        x_hbm.at[pl.ds(SIZE // 2, 1), pl.ds(SIZE // 2, 1), pl.ds(coff, HB)],
        cslab_v)

    runs = [[_sort_desc(cslab_v[0, 0, sid, pl.ds(c * 16, 16)])]
            for c in range(NCHUNK)]
    while len(runs) > 6:
        runs = [_merge(runs[2 * i], runs[2 * i + 1])
                for i in range(len(runs) // 2)]
    t01 = _merge(runs[0], runs[1], top_only=True)
    t23 = _merge(runs[2], runs[3], top_only=True)
    t45 = _merge(runs[4], runs[5], top_only=True)
    t = _merge(_merge(t01, t23, top_only=True), t45, top_only=True)
    for p in range(6):
        row_v[pl.ds(p * 16, 16)] = t[p]

    def sbody(k, _):
        s = start + k
        bsel = lax.rem(k, 2)
        pltpu.make_async_copy(
            x_hbm.at[pl.ds(0, 1), pl.ds(0, 1), pl.ds(coff, HB)],
            chunk_v.at[pl.ds(bsel, 1)], dma_sem).wait()

        @pl.when(k < NSLAB - 1)
        def _prefetch():
            s1 = s + 1
            pltpu.async_copy(
                x_hbm.at[pl.ds(s1 // SIZE, 1), pl.ds(lax.rem(s1, SIZE), 1),
                         pl.ds(coff, HB)],
                chunk_v.at[pl.ds(1 - bsel, 1)], dma_sem)

        @pl.when(k >= 2)
        def _drain():
            for r in range(2):
                pltpu.make_async_copy(
                    stage_v.at[pl.ds(bsel, 1), pl.ds(r, 1)],
                    shared_pix.at[pl.ds(r, 1), pl.ds(s, 1)],
                    out_sem).wait()

        for j in range(HB // NP):
            tops = _topk_chains(chunk_v, bsel, j * NP, NP)
            for q in range(NP):
                stage_v[bsel, j, pl.ds(q * 16, 16)] = tops[q]
        for r in range(2):
            pltpu.async_copy(stage_v.at[pl.ds(bsel, 1), pl.ds(r, 1)],
                             shared_pix.at[pl.ds(r, 1), pl.ds(s, 1)],
                             out_sem)
        return 0

    lax.fori_loop(0, NSLAB, sbody, 0)
    for _i in range(4):
        pltpu.make_async_copy(stage_v.at[pl.ds(0, 1), pl.ds(0, 1)],
                              shared_pix.at[pl.ds(0, 1), pl.ds(start, 1)],
                              out_sem).wait()

    q0 = lax.rem(sid, 4) * 4
    pltpu.make_async_copy(
        x_hbm.at[pl.ds(0, 1), pl.ds(0, 1), pl.ds(coff, HB)],
        xchunk_v, xdma_sem).wait()
    xtops = _topk_chains(xchunk_v, 0, q0, 4)
    for q in range(4):
        xstage_v[0, 0, pl.ds(q * 16, 16)] = xtops[q]
    pltpu.sync_copy(xstage_v, shared_extra.at[pl.ds(sid, 1)])

    plsc.subcore_barrier()

    pltpu.sync_copy(shared_pix.at[pl.ds(sid // 8, 1)], pieces_v)
    pltpu.sync_copy(shared_extra, xpieces_v)
    lane = lax.rem(sid, 8) * 16

    def abody(i, _):
        for d in range(4):
            s2 = i * 4 + d
            row_v[pl.ds(K_CEN + s2 * K_PIX, 16)] = pieces_v[
                0, s2, pl.ds(lane, 16)]
        return 0

    lax.fori_loop(0, 4 * NSLAB, abody, 0)
    for e in range(NXTRA):
        row_v[pl.ds(K_CEN + (16 * NSLAB + e) * K_PIX, 16)] = xpieces_v[
            e * 4 + sid // 4, 0, pl.ds(lax.rem(sid, 4) * 16, 16)]

    for ct in range(16):
        for l in range(8):
            row2_v[0, ct, 0, pl.ds(l * 16, 16)] = row_v[
                pl.ds(ct * 128 + l * 16, 16)]
    pltpu.sync_copy(
        row2_v,
        shared_row.at[pl.ds(sid // 8, 1), :, pl.ds(lax.rem(sid, 8), 1)])

    plsc.subcore_barrier()

    for k in range(2):
        tid = sid * 2 + k
        g_loc = tid // 16
        ct = lax.rem(tid, 16)
        pltpu.sync_copy(
            shared_row.at[pl.ds(g_loc, 1), pl.ds(ct, 1)],
            out4.at[pl.ds(cid * 2 + g_loc, 1), pl.ds(ct, 1)])


@jax.jit
def _run(x):
    mesh = plsc.VectorSubcoreMesh(core_axis_name="c", subcore_axis_name="s")
    fn = pl.kernel(
        _sc_body,
        out_type=jax.ShapeDtypeStruct((4, 16, 8, 128), jnp.float32),
        mesh=mesh,
        scratch_types=[
            pltpu.VMEM((2, 1, HB, CH), jnp.float32),
            pltpu.VMEM((1, 1, HB, CH), jnp.float32),
            pltpu.VMEM((1, 1, HB, CH), jnp.float32),
            pltpu.VMEM((2, 2, 128), jnp.float32),
            pltpu.VMEM((1, 1, 128), jnp.float32),
            pltpu.VMEM((1, PIX, 128), jnp.float32),
            pltpu.VMEM((16, 1, 128), jnp.float32),
            pltpu.VMEM((OUT_COLS + 16,), jnp.float32),
            pltpu.VMEM((1, 16, 1, 128), jnp.float32),
            pltpu.VMEM_SHARED((2, PIX, 128), jnp.float32),
            pltpu.VMEM_SHARED((2, 16, 8, 128), jnp.float32),
            pltpu.VMEM_SHARED((16, 1, 128), jnp.float32),
            pltpu.SemaphoreType.DMA,
            pltpu.SemaphoreType.DMA,
            pltpu.SemaphoreType.DMA,
        ],
        compiler_params=pltpu.CompilerParams(needs_layout_passes=False),
    )
    return fn(x)


def kernel(inputs):
    xt = inputs.transpose(1, 2, 0, 3)
    out4 = _run(xt)
    return out4.transpose(0, 2, 1, 3).reshape(BATCH, OUT_COLS)


# speedup vs baseline: 1.4503x; 1.4503x over previous
import jax
import jax.numpy as jnp
from jax import lax
from jax.experimental import pallas as pl
from jax.experimental.pallas import tpu as pltpu
from jax.experimental.pallas import tpu_sc as plsc

BATCH = 32
SIZE = 14
PIX = SIZE * SIZE
CH = 768
NCHUNK = CH // 16
K_PIX = 10
K_CEN = 88
OUT_COLS = K_CEN + PIX * K_PIX
NP = 8
HB = 16
NSLAB = PIX // 16
NXTRA = PIX - 16 * NSLAB


def _sort_desc(v):
    return plsc.sort_key_val(v, v, descending=True)[0]


def _bitonic_desc(c):
    n = len(c)
    if n == 1:
        return [_sort_desc(c[0])]
    h = [jnp.maximum(c[i], c[i + n // 2]) for i in range(n // 2)]
    l = [jnp.minimum(c[i], c[i + n // 2]) for i in range(n // 2)]
    return _bitonic_desc(h) + _bitonic_desc(l)


def _merge(a, b, top_only=False):
    n = len(a)
    rb = [lax.rev(v, (0,)) for v in reversed(b)]
    c = [jnp.maximum(a[i], rb[i]) for i in range(n)]
    top = _bitonic_desc(c)
    if top_only:
        return top
    d = [jnp.minimum(a[i], rb[i]) for i in range(n)]
    return top + _bitonic_desc(d)


def _topk_chains(chunk_v, bsel, b0, np_):
    neg = jnp.full((16,), -jnp.inf, jnp.float32)
    tops = [neg] * np_

    def _asc(c):
        return [
            lax.sort(chunk_v[bsel, 0, b0 + q, pl.ds(c * 16, 16)],
                     dimension=0)
            for q in range(np_)
        ]

    vas = _asc(0)
    nxt = _asc(1)
    for c in range(NCHUNK):
        nn = _asc(c + 2) if c + 2 < NCHUNK else None
        for q in range(np_):
            tops[q], _u = plsc.sort_key_val(
                jnp.maximum(tops[q], vas[q]), vas[q], descending=True)
        vas, nxt = nxt, nn
    return tops


def _sc_body(x_hbm, out4, chunk_v, xchunk_v, cslab_v, stage_v, xstage_v,
             pieces_v, xpieces_v, row_v, row2_v, shared_pix, shared_row,
             shared_extra, dma_sem, xdma_sem, out_sem):
    cid = lax.axis_index("c")
    sid = lax.axis_index("s")
    coff = cid * HB

    start = sid * NSLAB
    pltpu.async_copy(
        x_hbm.at[pl.ds(start // SIZE, 1), pl.ds(lax.rem(start, SIZE), 1),
                 pl.ds(coff, HB)],
        chunk_v.at[pl.ds(0, 1)], dma_sem)
    xslab = 16 * NSLAB + sid // 4
    pltpu.async_copy(
        x_hbm.at[pl.ds(xslab // SIZE, 1), pl.ds(lax.rem(xslab, SIZE), 1),
                 pl.ds(coff, HB)],
        xchunk_v, xdma_sem)
    pltpu.sync_copy(
        x_hbm.at[pl.ds(SIZE // 2, 1), pl.ds(SIZE // 2, 1), pl.ds(coff, HB)],
        cslab_v)

    runs = [[_sort_desc(cslab_v[0, 0, sid, pl.ds(c * 16, 16)])]
            for c in range(NCHUNK)]
    while len(runs) > 6:
        runs = [_merge(runs[2 * i], runs[2 * i + 1])
                for i in range(len(runs) // 2)]
    t01 = _merge(runs[0], runs[1], top_only=True)
    t23 = _merge(runs[2], runs[3], top_only=True)
    t45 = _merge(runs[4], runs[5], top_only=True)
    t = _merge(_merge(t01, t23, top_only=True), t45, top_only=True)
    for p in range(6):
        row_v[pl.ds(p * 16, 16)] = t[p]

    def sbody(k, _):
        s = start + k
        bsel = lax.rem(k, 2)
        pltpu.make_async_copy(
            x_hbm.at[pl.ds(0, 1), pl.ds(0, 1), pl.ds(coff, HB)],
            chunk_v.at[pl.ds(bsel, 1)], dma_sem).wait()

        @pl.when(k < NSLAB - 1)
        def _prefetch():
            s1 = s + 1
            pltpu.async_copy(
                x_hbm.at[pl.ds(s1 // SIZE, 1), pl.ds(lax.rem(s1, SIZE), 1),
                         pl.ds(coff, HB)],
                chunk_v.at[pl.ds(1 - bsel, 1)], dma_sem)

        @pl.when(k >= 2)
        def _drain():
            for r in range(2):
                pltpu.make_async_copy(
                    stage_v.at[pl.ds(bsel, 1), pl.ds(r, 1)],
                    shared_pix.at[pl.ds(r, 1), pl.ds(s, 1)],
                    out_sem).wait()

        def bbody(j, _):
            tops = _topk_chains(chunk_v, bsel, j * NP, NP)
            for q in range(NP):
                stage_v[bsel, j, pl.ds(q * 16, 16)] = tops[q]
            return 0

        lax.fori_loop(0, HB // NP, bbody, 0)
        for r in range(2):
            pltpu.async_copy(stage_v.at[pl.ds(bsel, 1), pl.ds(r, 1)],
                             shared_pix.at[pl.ds(r, 1), pl.ds(s, 1)],
                             out_sem)
        return 0

    lax.fori_loop(0, NSLAB, sbody, 0)
    for _i in range(4):
        pltpu.make_async_copy(stage_v.at[pl.ds(0, 1), pl.ds(0, 1)],
                              shared_pix.at[pl.ds(0, 1), pl.ds(start, 1)],
                              out_sem).wait()

    q0 = lax.rem(sid, 4) * 4
    pltpu.make_async_copy(
        x_hbm.at[pl.ds(0, 1), pl.ds(0, 1), pl.ds(coff, HB)],
        xchunk_v, xdma_sem).wait()
    xtops = _topk_chains(xchunk_v, 0, q0, 4)
    for q in range(4):
        xstage_v[0, 0, pl.ds(q * 16, 16)] = xtops[q]
    pltpu.sync_copy(xstage_v, shared_extra.at[pl.ds(sid, 1)])

    plsc.subcore_barrier()

    pltpu.sync_copy(shared_pix.at[pl.ds(sid // 8, 1)], pieces_v)
    pltpu.sync_copy(shared_extra, xpieces_v)
    lane = lax.rem(sid, 8) * 16

    def abody(i, _):
        for d in range(4):
            s2 = i * 4 + d
            row_v[pl.ds(K_CEN + s2 * K_PIX, 16)] = pieces_v[
                0, s2, pl.ds(lane, 16)]
        return 0

    lax.fori_loop(0, 4 * NSLAB, abody, 0)
    for e in range(NXTRA):
        row_v[pl.ds(K_CEN + (16 * NSLAB + e) * K_PIX, 16)] = xpieces_v[
            e * 4 + sid // 4, 0, pl.ds(lax.rem(sid, 4) * 16, 16)]

    for ct in range(16):
        for l in range(8):
            row2_v[0, ct, 0, pl.ds(l * 16, 16)] = row_v[
                pl.ds(ct * 128 + l * 16, 16)]
    pltpu.sync_copy(
        row2_v,
        shared_row.at[pl.ds(sid // 8, 1), :, pl.ds(lax.rem(sid, 8), 1)])

    plsc.subcore_barrier()

    for k in range(2):
        tid = sid * 2 + k
        g_loc = tid // 16
        ct = lax.rem(tid, 16)
        pltpu.sync_copy(
            shared_row.at[pl.ds(g_loc, 1), pl.ds(ct, 1)],
            out4.at[pl.ds(cid * 2 + g_loc, 1), pl.ds(ct, 1)])


@jax.jit
def _run(x):
    mesh = plsc.VectorSubcoreMesh(core_axis_name="c", subcore_axis_name="s")
    fn = pl.kernel(
        _sc_body,
        out_type=jax.ShapeDtypeStruct((4, 16, 8, 128), jnp.float32),
        mesh=mesh,
        scratch_types=[
            pltpu.VMEM((2, 1, HB, CH), jnp.float32),
            pltpu.VMEM((1, 1, HB, CH), jnp.float32),
            pltpu.VMEM((1, 1, HB, CH), jnp.float32),
            pltpu.VMEM((2, 2, 128), jnp.float32),
            pltpu.VMEM((1, 1, 128), jnp.float32),
            pltpu.VMEM((1, PIX, 128), jnp.float32),
            pltpu.VMEM((16, 1, 128), jnp.float32),
            pltpu.VMEM((OUT_COLS + 16,), jnp.float32),
            pltpu.VMEM((1, 16, 1, 128), jnp.float32),
            pltpu.VMEM_SHARED((2, PIX, 128), jnp.float32),
            pltpu.VMEM_SHARED((2, 16, 8, 128), jnp.float32),
            pltpu.VMEM_SHARED((16, 1, 128), jnp.float32),
            pltpu.SemaphoreType.DMA,
            pltpu.SemaphoreType.DMA,
            pltpu.SemaphoreType.DMA,
        ],
        compiler_params=pltpu.CompilerParams(needs_layout_passes=False),
    )
    return fn(x)


def kernel(inputs):
    xt = inputs.transpose(1, 2, 0, 3)
    out4 = _run(xt)
    return out4.transpose(0, 2, 1, 3).reshape(BATCH, OUT_COLS)


# speedup vs baseline: 1.4571x; 1.0047x over previous
import jax
import jax.numpy as jnp
from jax import lax
from jax.experimental import pallas as pl
from jax.experimental.pallas import tpu as pltpu
from jax.experimental.pallas import tpu_sc as plsc

BATCH = 32
SIZE = 14
PIX = SIZE * SIZE
CH = 768
NCHUNK = CH // 16
K_PIX = 10
K_CEN = 88
OUT_COLS = K_CEN + PIX * K_PIX
NP = 8
HB = 16
NSLAB = PIX // 16
NXTRA = PIX - 16 * NSLAB


def _sort_desc(v):
    return plsc.sort_key_val(v, v, descending=True)[0]


def _bitonic_desc(c):
    n = len(c)
    if n == 1:
        return [_sort_desc(c[0])]
    h = [jnp.maximum(c[i], c[i + n // 2]) for i in range(n // 2)]
    l = [jnp.minimum(c[i], c[i + n // 2]) for i in range(n // 2)]
    return _bitonic_desc(h) + _bitonic_desc(l)


def _merge(a, b, top_only=False):
    n = len(a)
    rb = [lax.rev(v, (0,)) for v in reversed(b)]
    c = [jnp.maximum(a[i], rb[i]) for i in range(n)]
    top = _bitonic_desc(c)
    if top_only:
        return top
    d = [jnp.minimum(a[i], rb[i]) for i in range(n)]
    return top + _bitonic_desc(d)


def _topk_chains(chunk_v, bsel, b0, np_):
    neg = jnp.full((16,), -jnp.inf, jnp.float32)
    tops = [neg] * np_

    def _asc(c):
        return [
            lax.sort(chunk_v[bsel, 0, b0 + q, pl.ds(c * 16, 16)],
                     dimension=0)
            for q in range(np_)
        ]

    vas = _asc(0)
    for c in range(NCHUNK):
        nxt = _asc(c + 1) if c + 1 < NCHUNK else None
        for q in range(np_):
            tops[q], _u = plsc.sort_key_val(
                jnp.maximum(tops[q], vas[q]), vas[q], descending=True)
        vas = nxt
    return tops


def _sc_body(x_hbm, out4, chunk_v, xchunk_v, cslab_v, stage_v, xstage_v,
             pieces_v, xpieces_v, row_v, row2_v, shared_pix, shared_row,
             shared_extra, dma_sem, xdma_sem, out_sem):
    cid = lax.axis_index("c")
    sid = lax.axis_index("s")
    coff = cid * HB

    start = sid * NSLAB
    pltpu.async_copy(
        x_hbm.at[pl.ds(start // SIZE, 1), pl.ds(lax.rem(start, SIZE), 1),
                 pl.ds(coff, HB)],
        chunk_v.at[pl.ds(0, 1)], dma_sem)
    xslab = 16 * NSLAB + sid // 4
    pltpu.async_copy(
        x_hbm.at[pl.ds(xslab // SIZE, 1), pl.ds(lax.rem(xslab, SIZE), 1),
                 pl.ds(coff, HB)],
        xchunk_v, xdma_sem)
    pltpu.sync_copy(
        x_hbm.at[pl.ds(SIZE // 2, 1), pl.ds(SIZE // 2, 1), pl.ds(coff, HB)],
        cslab_v)

    runs = [[_sort_desc(cslab_v[0, 0, sid, pl.ds(c * 16, 16)])]
            for c in range(NCHUNK)]
    while len(runs) > 6:
        runs = [_merge(runs[2 * i], runs[2 * i + 1])
                for i in range(len(runs) // 2)]
    t01 = _merge(runs[0], runs[1], top_only=True)
    t23 = _merge(runs[2], runs[3], top_only=True)
    t45 = _merge(runs[4], runs[5], top_only=True)
    t = _merge(_merge(t01, t23, top_only=True), t45, top_only=True)
    for p in range(6):
        row_v[pl.ds(p * 16, 16)] = t[p]

    def sbody(k, _):
        s = start + k
        bsel = lax.rem(k, 2)
        pltpu.make_async_copy(
            x_hbm.at[pl.ds(0, 1), pl.ds(0, 1), pl.ds(coff, HB)],
            chunk_v.at[pl.ds(bsel, 1)], dma_sem).wait()

        @pl.when(k < NSLAB - 1)
        def _prefetch():
            s1 = s + 1
            pltpu.async_copy(
                x_hbm.at[pl.ds(s1 // SIZE, 1), pl.ds(lax.rem(s1, SIZE), 1),
                         pl.ds(coff, HB)],
                chunk_v.at[pl.ds(1 - bsel, 1)], dma_sem)

        @pl.when(k >= 2)
        def _drain():
            for r in range(2):
                pltpu.make_async_copy(
                    stage_v.at[pl.ds(bsel, 1), pl.ds(r, 1)],
                    shared_pix.at[pl.ds(r, 1), pl.ds(s, 1)],
                    out_sem).wait()

        def bbody(j, _):
            tops = _topk_chains(chunk_v, bsel, j * NP, NP)
            for q in range(NP):
                stage_v[bsel, j, pl.ds(q * 16, 16)] = tops[q]
            return 0

        lax.fori_loop(0, HB // NP, bbody, 0)
        for r in range(2):
            pltpu.async_copy(stage_v.at[pl.ds(bsel, 1), pl.ds(r, 1)],
                             shared_pix.at[pl.ds(r, 1), pl.ds(s, 1)],
                             out_sem)
        return 0

    lax.fori_loop(0, NSLAB, sbody, 0)
    for _i in range(4):
        pltpu.make_async_copy(stage_v.at[pl.ds(0, 1), pl.ds(0, 1)],
                              shared_pix.at[pl.ds(0, 1), pl.ds(start, 1)],
                              out_sem).wait()

    q0 = lax.rem(sid, 4) * 4
    pltpu.make_async_copy(
        x_hbm.at[pl.ds(0, 1), pl.ds(0, 1), pl.ds(coff, HB)],
        xchunk_v, xdma_sem).wait()
    xtops = _topk_chains(xchunk_v, 0, q0, 4)
    for q in range(4):
        xstage_v[0, 0, pl.ds(q * 16, 16)] = xtops[q]
    pltpu.sync_copy(xstage_v, shared_extra.at[pl.ds(sid, 1)])

    plsc.subcore_barrier()

    pltpu.sync_copy(shared_pix.at[pl.ds(sid // 8, 1)], pieces_v)
    pltpu.sync_copy(shared_extra, xpieces_v)
    lane = lax.rem(sid, 8) * 16

    def abody(i, _):
        for d in range(4):
            s2 = i * 4 + d
            row_v[pl.ds(K_CEN + s2 * K_PIX, 16)] = pieces_v[
                0, s2, pl.ds(lane, 16)]
        return 0

    lax.fori_loop(0, 4 * NSLAB, abody, 0)
    for e in range(NXTRA):
        row_v[pl.ds(K_CEN + (16 * NSLAB + e) * K_PIX, 16)] = xpieces_v[
            e * 4 + sid // 4, 0, pl.ds(lax.rem(sid, 4) * 16, 16)]

    for ct in range(16):
        for l in range(8):
            row2_v[0, ct, 0, pl.ds(l * 16, 16)] = row_v[
                pl.ds(ct * 128 + l * 16, 16)]
    pltpu.sync_copy(
        row2_v,
        shared_row.at[pl.ds(sid // 8, 1), :, pl.ds(lax.rem(sid, 8), 1)])

    plsc.subcore_barrier()

    for k in range(2):
        tid = sid * 2 + k
        g_loc = tid // 16
        ct = lax.rem(tid, 16)
        pltpu.sync_copy(
            shared_row.at[pl.ds(g_loc, 1), pl.ds(ct, 1)],
            out4.at[pl.ds(cid * 2 + g_loc, 1), pl.ds(ct, 1)])


@jax.jit
def _run(x):
    mesh = plsc.VectorSubcoreMesh(core_axis_name="c", subcore_axis_name="s")
    fn = pl.kernel(
        _sc_body,
        out_type=jax.ShapeDtypeStruct((4, 16, 8, 128), jnp.float32),
        mesh=mesh,
        scratch_types=[
            pltpu.VMEM((2, 1, HB, CH), jnp.float32),
            pltpu.VMEM((1, 1, HB, CH), jnp.float32),
            pltpu.VMEM((1, 1, HB, CH), jnp.float32),
            pltpu.VMEM((2, 2, 128), jnp.float32),
            pltpu.VMEM((1, 1, 128), jnp.float32),
            pltpu.VMEM((1, PIX, 128), jnp.float32),
            pltpu.VMEM((16, 1, 128), jnp.float32),
            pltpu.VMEM((OUT_COLS + 16,), jnp.float32),
            pltpu.VMEM((1, 16, 1, 128), jnp.float32),
            pltpu.VMEM_SHARED((2, PIX, 128), jnp.float32),
            pltpu.VMEM_SHARED((2, 16, 8, 128), jnp.float32),
            pltpu.VMEM_SHARED((16, 1, 128), jnp.float32),
            pltpu.SemaphoreType.DMA,
            pltpu.SemaphoreType.DMA,
            pltpu.SemaphoreType.DMA,
        ],
        compiler_params=pltpu.CompilerParams(needs_layout_passes=False),
    )
    return fn(x)


def kernel(inputs):
    xt = inputs.transpose(1, 2, 0, 3)
    out4 = _run(xt)
    return out4.transpose(0, 2, 1, 3).reshape(BATCH, OUT_COLS)
